# Initial kernel scaffold; baseline (speedup 1.0000x reference)
import jax
import jax.numpy as jnp
from jax.experimental import pallas as pl


def kernel(x, edge_index, W1, b1, W2, b2):
    raise NotImplementedError("write your pallas kernel here")


# speedup vs baseline: 16.9067x; 16.9067x over previous
import functools

import jax
import jax.numpy as jnp
from jax import lax
from jax.experimental import pallas as pl
from jax.experimental.pallas import tpu as pltpu
from jax.experimental.pallas import tpu_sc as plsc

NC = 2
NS = 16
NW = NC * NS
LANES = 16
K = 128


def _zero_rows(ref, n_rows, n_cols):
    zeros16 = jnp.zeros((LANES,), jnp.float32)

    @pl.loop(0, n_rows)
    def _(i):
        for j in range(n_cols // LANES):
            ref[i, pl.ds(j * LANES, LANES)] = zeros16


def _fill_1d(ref, n, value):
    vec = jnp.full((LANES,), value, jnp.float32)
    for j in range(n // LANES):
        ref[pl.ds(j * LANES, LANES)] = vec


def _make_deg_kernel(e_total, n_pad):
    e_w = e_total // NW
    n_full = e_w // K
    rem = e_w - n_full * K
    rows_per_tile = n_pad // NS
    mesh = plsc.VectorSubcoreMesh(core_axis_name="c", subcore_axis_name="s")

    @functools.partial(
        pl.kernel,
        out_type=jax.ShapeDtypeStruct((NC, n_pad), jnp.float32),
        mesh=mesh,
        scratch_types=[
            pltpu.VMEM((K,), jnp.int32),
            pltpu.VMEM((rem,), jnp.int32),
            pltpu.VMEM((K,), jnp.float32),
            pltpu.VMEM((K,), jnp.float32),
            pltpu.VMEM_SHARED((n_pad,), jnp.float32),
        ],
    )
    def deg_kernel(dst_hbm, out_hbm, dst_v, dst_r, ones_v, zero_v, acc_sh):
        c = lax.axis_index("c")
        s = lax.axis_index("s")
        wid = s * NC + c
        _fill_1d(ones_v, K, 1.0)
        _fill_1d(zero_v, K, 0.0)
        for j in range(rows_per_tile // K):
            pltpu.sync_copy(zero_v, acc_sh.at[pl.ds(s * rows_per_tile + j * K, K)])
        plsc.subcore_barrier()

        base = wid * e_w

        @pl.loop(0, n_full)
        def _(i):
            pltpu.sync_copy(dst_hbm.at[pl.ds(base + i * K, K)], dst_v)
            pltpu.sync_copy(ones_v, acc_sh.at[dst_v], add=True)

        if rem:
            pltpu.sync_copy(dst_hbm.at[pl.ds(base + n_full * K, rem)], dst_r)
            pltpu.sync_copy(ones_v.at[pl.ds(0, rem)], acc_sh.at[dst_r], add=True)

        plsc.subcore_barrier()
        pltpu.sync_copy(acc_sh.at[pl.ds(s * rows_per_tile, rows_per_tile)],
                        out_hbm.at[c, pl.ds(s * rows_per_tile, rows_per_tile)])

    return deg_kernel


def _make_scatter_kernel(n_nodes, e_total, n_pad, d):
    e_w = e_total // NW
    n_full = e_w // K
    rem = e_w - n_full * K
    rows_per_tile = n_pad // NS
    mesh = plsc.VectorSubcoreMesh(core_axis_name="c", subcore_axis_name="s")

    @functools.partial(
        pl.kernel,
        out_type=jax.ShapeDtypeStruct((NC, n_pad, d), jnp.float32),
        mesh=mesh,
        scratch_types=[
            pltpu.VMEM((K,), jnp.int32),
            pltpu.VMEM((K,), jnp.int32),
            pltpu.VMEM((rem,), jnp.int32),
            pltpu.VMEM((rem,), jnp.int32),
            pltpu.VMEM((K, d), jnp.float32),
            pltpu.VMEM((rem, d), jnp.float32),
            pltpu.VMEM_SHARED((n_pad, d), jnp.float32),
            pltpu.SemaphoreType.DMA,
        ],
    )
    def scatter_kernel(hs_hbm, src_hbm, dst_hbm, out_hbm,
                       src_v, dst_v, src_r, dst_r, rows_v, rows_r, acc_sh, sem):
        c = lax.axis_index("c")
        s = lax.axis_index("s")
        wid = s * NC + c

        _zero_rows(rows_v, K, d)
        for j in range(rows_per_tile // K):
            pltpu.sync_copy(rows_v, acc_sh.at[pl.ds(s * rows_per_tile + j * K, K)])
        plsc.subcore_barrier()

        base = wid * e_w

        @pl.loop(0, n_full)
        def _(i):
            pltpu.sync_copy(src_hbm.at[pl.ds(base + i * K, K)], src_v)
            pltpu.sync_copy(dst_hbm.at[pl.ds(base + i * K, K)], dst_v)
            pltpu.async_copy(hs_hbm.at[src_v], rows_v, sem).wait()
            pltpu.sync_copy(rows_v, acc_sh.at[dst_v], add=True)

        if rem:
            off = base + n_full * K
            pltpu.sync_copy(src_hbm.at[pl.ds(off, rem)], src_r)
            pltpu.sync_copy(dst_hbm.at[pl.ds(off, rem)], dst_r)
            pltpu.async_copy(hs_hbm.at[src_r], rows_r, sem).wait()
            pltpu.sync_copy(rows_r, acc_sh.at[dst_r], add=True)

        plsc.subcore_barrier()
        pltpu.sync_copy(acc_sh.at[pl.ds(s * rows_per_tile, rows_per_tile)],
                        out_hbm.at[c, pl.ds(s * rows_per_tile, rows_per_tile)])

    return scatter_kernel


def _dinv_block(degp_ref):
    deg = degp_ref[0, :] + degp_ref[1, :] + 1.0
    return lax.rsqrt(deg)


def _tc_first(x, w1, degp, r):
    n, d_in = x.shape
    d_h = w1.shape[1]

    def body(x_ref, w_ref, degp_ref, o_ref):
        h = jnp.dot(x_ref[...], w_ref[...], preferred_element_type=jnp.float32)
        o_ref[...] = h * _dinv_block(degp_ref)[:, None]

    return pl.pallas_call(
        body,
        grid=(pl.cdiv(n, r),),
        in_specs=[
            pl.BlockSpec((r, d_in), lambda i: (i, 0)),
            pl.BlockSpec((d_in, d_h), lambda i: (0, 0)),
            pl.BlockSpec((NC, r), lambda i: (0, i)),
        ],
        out_specs=pl.BlockSpec((r, d_h), lambda i: (i, 0)),
        out_shape=jax.ShapeDtypeStruct((n, d_h), jnp.float32),
    )(x, w1, degp)


def _tc_mid(s1, hs1, degp, b1, w2, r):
    n, d_h = hs1.shape
    d_o = w2.shape[1]

    def body(s1_ref, hs_ref, degp_ref, b_ref, w_ref, o_ref):
        dinv = _dinv_block(degp_ref)[:, None]
        conv = dinv * (s1_ref[0] + s1_ref[1] + hs_ref[...]) + b_ref[...]
        h = jnp.maximum(conv, 0.0)
        o_ref[...] = jnp.dot(h, w_ref[...], preferred_element_type=jnp.float32) * dinv

    return pl.pallas_call(
        body,
        grid=(pl.cdiv(n, r),),
        in_specs=[
            pl.BlockSpec((NC, r, d_h), lambda i: (0, i, 0)),
            pl.BlockSpec((r, d_h), lambda i: (i, 0)),
            pl.BlockSpec((NC, r), lambda i: (0, i)),
            pl.BlockSpec((1, d_h), lambda i: (0, 0)),
            pl.BlockSpec((d_h, d_o), lambda i: (0, 0)),
        ],
        out_specs=pl.BlockSpec((r, d_o), lambda i: (i, 0)),
        out_shape=jax.ShapeDtypeStruct((n, d_o), jnp.float32),
    )(s1, hs1, degp, b1, w2)


def _tc_final(s2, hs2, degp, b2, r):
    n, d_o = hs2.shape

    def body(s2_ref, hs_ref, degp_ref, b_ref, o_ref):
        dinv = _dinv_block(degp_ref)[:, None]
        conv = dinv * (s2_ref[0] + s2_ref[1] + hs_ref[...]) + b_ref[...]
        o_ref[...] = jnp.maximum(conv, 0.0)

    return pl.pallas_call(
        body,
        grid=(pl.cdiv(n, r),),
        in_specs=[
            pl.BlockSpec((NC, r, d_o), lambda i: (0, i, 0)),
            pl.BlockSpec((r, d_o), lambda i: (i, 0)),
            pl.BlockSpec((NC, r), lambda i: (0, i)),
            pl.BlockSpec((1, d_o), lambda i: (0, 0)),
        ],
        out_specs=pl.BlockSpec((r, d_o), lambda i: (i, 0)),
        out_shape=jax.ShapeDtypeStruct((n, d_o), jnp.float32),
    )(s2, hs2, degp, b2)


def kernel(x, edge_index, W1, b1, W2, b2):
    n, d_in = x.shape
    e_total = edge_index.shape[1]
    d_h = W1.shape[1]
    d_o = W2.shape[1]
    n_pad = ((n + NW * LANES - 1) // (NW * LANES)) * (NW * LANES)

    src = edge_index[0].astype(jnp.int32)
    dst = edge_index[1].astype(jnp.int32)
    b1r = b1.reshape(1, d_h).astype(jnp.float32)
    b2r = b2.reshape(1, d_o).astype(jnp.float32)

    r = 1024

    degp = _make_deg_kernel(e_total, n_pad)(dst)
    scat = _make_scatter_kernel(n, e_total, n_pad, d_h)

    hs1 = _tc_first(x, W1.astype(jnp.float32), degp, r)
    s1 = scat(hs1, src, dst)
    hs2 = _tc_mid(s1, hs1, degp, b1r, W2.astype(jnp.float32), r)
    s2 = scat(hs2, src, dst)
    out = _tc_final(s2, hs2, degp, b2r, r)
    return out
